# baseline (speedup 1.0000x reference)
import functools

import jax
import jax.numpy as jnp
from jax.experimental import pallas as pl
from jax.experimental.pallas import tpu as pltpu

KP = 15


def _lrelu(x):
    return jax.nn.leaky_relu(x, 0.1)


def _kpconv(q_pts, s_pts, neighb, x, kpts, w, sigma):
    neighb_x = x[neighb]
    rel = s_pts[neighb] - q_pts[:, None, :]
    sq = jnp.sum((rel[:, :, None, :] - kpts[None, None, :, :]) ** 2, axis=-1)
    infl = jnp.maximum(0.0, 1.0 - jnp.sqrt(sq + 1e-12) / sigma)
    weighted = jnp.einsum('nkp,nkc->npc', infl, neighb_x)
    return jnp.einsum('npc,pco->no', weighted, w)


def _resnet(x, q_pts, s_pts, neighb, kpts, sigma, p, strided):
    y = _lrelu(x @ p['w1'] + p['b1'])
    y = _lrelu(_kpconv(q_pts, s_pts, neighb, y, kpts, p['wc'], sigma) + p['bc'])
    y = y @ p['w2'] + p['b2']
    sc = jnp.max(x[neighb], axis=1) if strided else x
    if 'ws' in p:
        sc = sc @ p['ws'] + p['bs']
    return _lrelu(y + sc)


def _matmul_bias_kernel(x_ref, w_ref, b_ref, o_ref):
    o_ref[...] = jnp.dot(x_ref[...], w_ref[...],
                         preferred_element_type=jnp.float32) + b_ref[...]


def _final_linear(x, w, b):
    n, cin = x.shape
    cout = w.shape[1]
    bn = 2000
    return pl.pallas_call(
        _matmul_bias_kernel,
        grid=(n // bn,),
        in_specs=[
            pl.BlockSpec((bn, cin), lambda i: (i, 0)),
            pl.BlockSpec((cin, cout), lambda i: (0, 0)),
            pl.BlockSpec((1, cout), lambda i: (0, 0)),
        ],
        out_specs=pl.BlockSpec((bn, cout), lambda i: (i, 0)),
        out_shape=jax.ShapeDtypeStruct((n, cout), jnp.float32),
    )(x, w, b.reshape(1, -1))


def kernel(features, points0, points1, points2, params, neighbors0,
           neighbors1, neighbors2, pools0, pools1, upsamples0, upsamples1):
    kp0 = params['kp']
    kp1 = kp0 * 2.0
    kp2 = kp0 * 4.0
    s0, s1, s2 = 0.3, 0.6, 1.2
    x = _lrelu(_kpconv(points0, points0, neighbors0, features, kp0,
                       params['b0']['w'], s0) + params['b0']['b'])
    x = _resnet(x, points0, points0, neighbors0, kp0, s0, params['b1'], False)
    skip0 = x
    x = _resnet(x, points1, points0, pools0, kp0, s0, params['b2'], True)
    x = _resnet(x, points1, points1, neighbors1, kp1, s1, params['b3'], False)
    skip1 = x
    x = _resnet(x, points2, points1, pools1, kp1, s1, params['b4'], True)
    x = _resnet(x, points2, points2, neighbors2, kp2, s2, params['b5'], False)
    x = x[upsamples1[:, 0]]
    x = jnp.concatenate([x, skip1], axis=1)
    x = _lrelu(x @ params['d1']['w'] + params['d1']['b'])
    x = x[upsamples0[:, 0]]
    x = jnp.concatenate([x, skip0], axis=1)
    return _final_linear(x, params['d3']['w'], params['d3']['b'])


# speedup vs baseline: 4.2467x; 4.2467x over previous
import functools
import math

import jax
import jax.numpy as jnp
from jax import lax
from jax.experimental import pallas as pl
from jax.experimental.pallas import tpu as pltpu
from jax.experimental.pallas import tpu_sc as plsc

K = 16
KP = 15
N0, N1, N2 = 50000, 12500, 3125
N0P, N1P, N2P = 50176, 12544, 3200
NW = 32


def _lrelu(x):
    return jnp.where(x >= 0, x, 0.1 * x)


@functools.cache
def _tgather_fn(ns, ct, cu, kk, np_, r):
    nchunks = np_ // r
    cpw = -(-nchunks // NW)
    jblocks = r // 16

    mesh = plsc.VectorSubcoreMesh(core_axis_name="c", subcore_axis_name="s")

    @functools.partial(
        pl.kernel,
        out_type=jax.ShapeDtypeStruct((kk, cu, np_), jnp.float32),
        mesh=mesh,
        scratch_types=[
            pltpu.VMEM((r * kk,), jnp.int32),
            pltpu.VMEM((r * kk, ct), jnp.float32),
            pltpu.VMEM((kk, cu, r), jnp.float32),
            pltpu.SemaphoreType.DMA,
        ],
        compiler_params=pltpu.CompilerParams(
            use_tc_tiling_on_sc=False, needs_layout_passes=False),
    )
    def tg(table_hbm, idx_hbm, out_hbm, idx_v, rows_v, obuf, sem):
        wid = lax.axis_index("s") * 2 + lax.axis_index("c")
        lane = lax.iota(jnp.int32, 16)

        def chunk_body(t, _):
            cid = wid * cpw + t

            @pl.when(cid < nchunks)
            def _():
                n0 = cid * r
                pltpu.sync_copy(idx_hbm.at[pl.ds(n0 * kk, r * kk)], idx_v)
                pltpu.async_copy(table_hbm.at[idx_v], rows_v, sem).wait()

                def c_body(c, _):
                    cvec = jnp.full((16,), 0, jnp.int32) + c

                    def j_body(jb, _):
                        rbase = lane * kk + jb * (16 * kk)
                        for k in range(kk):
                            v = plsc.load_gather(rows_v, [rbase + k, cvec])
                            obuf[k, c, pl.ds(jb * 16, 16)] = v
                        return 0

                    lax.fori_loop(0, jblocks, j_body, 0, unroll=False)
                    return 0

                lax.fori_loop(0, cu, c_body, 0, unroll=False)
                pltpu.sync_copy(obuf, out_hbm.at[:, :, pl.ds(n0, r)])

            return 0

        lax.fori_loop(0, cpw, chunk_body, 0, unroll=False)

    return tg


def _tgather(table, idx_flat, cu, kk):
    ns, ct = table.shape
    np_ = idx_flat.shape[0] // kk
    budget = 384 * 1024
    r = 16
    for cand in (128, 64, 32, 16):
        if (cand * kk * ct + kk * cu * cand) * 4 <= budget and np_ % cand == 0:
            r = cand
            break
    return _tgather_fn(ns, ct, cu, kk, np_, r)(table, idx_flat)


@functools.cache
def _rgather_fn(v, d, bp, rb):
    nchunks = bp // rb
    cpw = -(-nchunks // NW)
    mesh = plsc.VectorSubcoreMesh(core_axis_name="c", subcore_axis_name="s")

    @functools.partial(
        pl.kernel,
        out_type=jax.ShapeDtypeStruct((bp, d), jnp.float32),
        mesh=mesh,
        scratch_types=[
            pltpu.VMEM((rb,), jnp.int32),
            pltpu.VMEM((rb, d), jnp.float32),
            pltpu.SemaphoreType.DMA,
        ],
        compiler_params=pltpu.CompilerParams(
            use_tc_tiling_on_sc=False, needs_layout_passes=False),
    )
    def rg(table_hbm, idx_hbm, out_hbm, idx_v, rows_v, sem):
        wid = lax.axis_index("s") * 2 + lax.axis_index("c")

        def chunk_body(t, _):
            cid = wid * cpw + t

            @pl.when(cid < nchunks)
            def _():
                n0 = cid * rb
                pltpu.sync_copy(idx_hbm.at[pl.ds(n0, rb)], idx_v)
                pltpu.async_copy(table_hbm.at[idx_v], rows_v, sem).wait()
                pltpu.sync_copy(rows_v, out_hbm.at[pl.ds(n0, rb)])

            return 0

        lax.fori_loop(0, cpw, chunk_body, 0, unroll=False)

    return rg


def _rgather(table, idx):
    v, d = table.shape
    bp = idx.shape[0]
    rb = 256 if d > 128 else 512
    while bp % rb:
        rb //= 2
    return _rgather_fn(v, d, bp, rb)(table, idx)


def _kpconv_call(mode, gt, qt, kpd, kpt2, wflat, bc, sigma, c, o,
                 extras, np_, bn, cout):
    cu = gt.shape[1]
    if mode == "b0":
        poff, feat_off = 0, 3
    elif mode == "ws":
        poff = c
    else:
        cs = extras[2].shape[0]
        poff = c + cs

    def body(*refs):
        if mode == "b0":
            gt_ref, qt_ref, kpd_ref, kpt2_ref, wflat_ref, bc_ref, o_ref = refs
        elif mode == "ws":
            (gt_ref, qt_ref, kpd_ref, kpt2_ref, wflat_ref, bc_ref,
             x_ref, w2_ref, b2_ref, ws_ref, bs_ref, o_ref) = refs
        else:
            (gt_ref, qt_ref, kpd_ref, kpt2_ref, wflat_ref, bc_ref,
             w2_ref, b2_ref, eye_ref, o_ref) = refs

        gt_b = gt_ref[...]
        qt_b = qt_ref[...]
        pt = gt_b[:, poff:poff + 3, :]
        rel = pt - qt_b[None, :, :]
        relf = rel.reshape(K * 3, bn)
        rk = lax.dot_general(kpd_ref[...], relf,
                             (((1,), (0,)), ((), ())),
                             preferred_element_type=jnp.float32)
        rel2 = jnp.sum(rel * rel, axis=1)
        rel2t = jnp.broadcast_to(rel2[:, None, :],
                                 (K, KP, bn)).reshape(K * KP, bn)
        sq = rel2t - 2.0 * rk + kpt2_ref[...]
        infl = jnp.maximum(
            0.0, 1.0 - jnp.sqrt(jnp.maximum(sq, 0.0) + 1e-12) / sigma)

        if mode == "b0":
            yg = gt_b[:, feat_off:feat_off + 1, :]
        else:
            yg = gt_b[:, 0:c, :]
        rows = []
        for p in range(KP):
            acc = infl[p, :][None, :] * yg[0]
            for k in range(1, K):
                acc = acc + infl[k * KP + p, :][None, :] * yg[k]
            rows.append(acc)
        wt = jnp.concatenate(rows, axis=0)
        y = lax.dot_general(wt, wflat_ref[...],
                            (((0,), (0,)), ((), ())),
                            preferred_element_type=jnp.float32)
        y = _lrelu(y + bc_ref[...])
        if mode == "b0":
            o_ref[...] = y
            return
        y = jnp.dot(y, w2_ref[...],
                    preferred_element_type=jnp.float32) + b2_ref[...]
        if mode == "ws":
            scp = jnp.dot(x_ref[...], ws_ref[...],
                          preferred_element_type=jnp.float32) + bs_ref[...]
        else:
            xg = gt_b[:, c:c + cs, :]
            sct = jnp.max(xg, axis=0)
            scp = lax.dot_general(sct, eye_ref[...],
                                  (((0,), (0,)), ((), ())),
                                  preferred_element_type=jnp.float32)
        o_ref[...] = _lrelu(y + scp)

    cc = 1 if mode == "b0" else c
    in_specs = [
        pl.BlockSpec((K, cu, bn), lambda i: (0, 0, i)),
        pl.BlockSpec((3, bn), lambda i: (0, i)),
        pl.BlockSpec((K * KP, K * 3), lambda i: (0, 0)),
        pl.BlockSpec((K * KP, 1), lambda i: (0, 0)),
        pl.BlockSpec((KP * cc, o), lambda i: (0, 0)),
        pl.BlockSpec((1, o), lambda i: (0, 0)),
    ]
    args = [gt, qt, kpd, kpt2, wflat, bc]
    if mode == "ws":
        x, w2, b2, ws, bs = extras
        cin = x.shape[1]
        in_specs += [
            pl.BlockSpec((bn, cin), lambda i: (i, 0)),
            pl.BlockSpec((o, cout), lambda i: (0, 0)),
            pl.BlockSpec((1, cout), lambda i: (0, 0)),
            pl.BlockSpec((cin, cout), lambda i: (0, 0)),
            pl.BlockSpec((1, cout), lambda i: (0, 0)),
        ]
        args += [x, w2, b2.reshape(1, -1), ws, bs.reshape(1, -1)]
    elif mode == "pool":
        w2, b2, eye_cs = extras
        in_specs += [
            pl.BlockSpec((o, cout), lambda i: (0, 0)),
            pl.BlockSpec((1, cout), lambda i: (0, 0)),
            pl.BlockSpec((cs, cs), lambda i: (0, 0)),
        ]
        args += [w2, b2.reshape(1, -1), eye_cs]

    return pl.pallas_call(
        body,
        grid=(np_ // bn,),
        in_specs=in_specs,
        out_specs=pl.BlockSpec((bn, cout), lambda i: (i, 0)),
        out_shape=jax.ShapeDtypeStruct((np_, cout), jnp.float32),
    )(*args)


def _unary_call(xs, ws, b, act, bn):
    np_ = xs[0].shape[0]
    o = ws[0].shape[1]

    def body(*refs):
        n_in = len(xs)
        acc = refs[2 * n_in][...]
        for i in range(n_in):
            acc = acc + jnp.dot(refs[i][...], refs[n_in + i][...],
                                preferred_element_type=jnp.float32)
        refs[-1][...] = _lrelu(acc) if act else acc

    in_specs = [pl.BlockSpec((bn, x.shape[1]), lambda i: (i, 0)) for x in xs]
    in_specs += [pl.BlockSpec(w.shape, lambda i: (0, 0)) for w in ws]
    in_specs += [pl.BlockSpec((1, o), lambda i: (0, 0))]
    return pl.pallas_call(
        body,
        grid=(np_ // bn,),
        in_specs=in_specs,
        out_specs=pl.BlockSpec((bn, o), lambda i: (i, 0)),
        out_shape=jax.ShapeDtypeStruct((np_, o), jnp.float32),
    )(*xs, *ws, b.reshape(1, -1))


def _pad_rows(a, n):
    return jnp.pad(a, ((0, n - a.shape[0]),) + ((0, 0),) * (a.ndim - 1))


def _mktable(parts, ctot):
    t = jnp.concatenate(parts, axis=1)
    return jnp.pad(t, ((0, 0), (0, ctot - t.shape[1])))


def kernel(features, points0, points1, points2, params, neighbors0,
           neighbors1, neighbors2, pools0, pools1, upsamples0, upsamples1):
    p = params
    kp0 = p['kp']

    nb0 = _pad_rows(neighbors0, N0P).reshape(-1)
    nb1 = _pad_rows(neighbors1, N1P).reshape(-1)
    nb2 = _pad_rows(neighbors2, N2P).reshape(-1)
    pl0 = _pad_rows(pools0, N1P).reshape(-1)
    pl1 = _pad_rows(pools1, N2P).reshape(-1)
    up0 = _pad_rows(upsamples0[:, 0], N0P)
    up1 = _pad_rows(upsamples1[:, 0], N1P)
    ia0 = jnp.arange(N0P, dtype=jnp.int32)
    ia1 = jnp.arange(N1P, dtype=jnp.int32)
    ia2 = jnp.arange(N2P, dtype=jnp.int32)

    pts0 = _pad_rows(points0, N0P)
    pts1 = _pad_rows(points1, N1P)
    pts2 = _pad_rows(points2, N2P)
    feat = _pad_rows(features, N0P)

    eye16 = jnp.eye(K, dtype=jnp.float32)
    consts = []
    for lvl in range(3):
        kpts = kp0 * (2.0 ** lvl)
        sig = 0.3 * (2.0 ** lvl)
        kpd = jnp.kron(eye16, kpts)
        kpt2 = jnp.tile(jnp.sum(kpts * kpts, axis=1), K).reshape(-1, 1)
        consts.append((kpd, kpt2, sig))

    def wflat(wc):
        return wc.reshape(KP * wc.shape[1], wc.shape[2])

    bn0, bn1, bn2 = 896, 896, 640

    t0 = _mktable([pts0, feat], 16)
    qt0 = _tgather(t0, ia0, 3, 1).reshape(3, N0P)
    gt_b0 = _tgather(t0, nb0, 4, K)
    kpd0, kpt20, s0 = consts[0]
    x0 = _kpconv_call("b0", gt_b0, qt0, kpd0, kpt20,
                      p['b0']['w'].reshape(KP, 32), p['b0']['b'].reshape(1, -1),
                      s0, 1, 32, None, N0P, bn0, 32)

    y1 = _unary_call([x0], [p['b1']['w1']], p['b1']['b1'], True, bn0)
    gt1 = _tgather(_mktable([y1, pts0], 32), nb0, 19, K)
    x1 = _kpconv_call("ws", gt1, qt0, kpd0, kpt20, wflat(p['b1']['wc']),
                      p['b1']['bc'].reshape(1, -1), s0, 16, 16,
                      (x0, p['b1']['w2'], p['b1']['b2'], p['b1']['ws'],
                       p['b1']['bs']), N0P, bn0, 64)

    y2 = _unary_call([x1], [p['b2']['w1']], p['b2']['b1'], True, bn0)
    qt1 = _tgather(_mktable([pts1], 16), ia1, 3, 1).reshape(3, N1P)
    gt2 = _tgather(_mktable([y2, x1, pts0], 96), pl0, 83, K)
    x2 = _kpconv_call("pool", gt2, qt1, kpd0, kpt20, wflat(p['b2']['wc']),
                      p['b2']['bc'].reshape(1, -1), s0, 16, 16,
                      (p['b2']['w2'], p['b2']['b2'],
                       jnp.eye(64, dtype=jnp.float32)), N1P, bn1, 64)

    y3 = _unary_call([x2], [p['b3']['w1']], p['b3']['b1'], True, bn1)
    gt3 = _tgather(_mktable([y3, pts1], 48), nb1, 35, K)
    kpd1, kpt21, s1 = consts[1]
    x3 = _kpconv_call("ws", gt3, qt1, kpd1, kpt21, wflat(p['b3']['wc']),
                      p['b3']['bc'].reshape(1, -1), s1, 32, 32,
                      (x2, p['b3']['w2'], p['b3']['b2'], p['b3']['ws'],
                       p['b3']['bs']), N1P, bn1, 128)

    y4 = _unary_call([x3], [p['b4']['w1']], p['b4']['b1'], True, bn1)
    qt2 = _tgather(_mktable([pts2], 16), ia2, 3, 1).reshape(3, N2P)
    gt4 = _tgather(_mktable([y4, x3, pts1], 176), pl1, 163, K)
    x4 = _kpconv_call("pool", gt4, qt2, kpd1, kpt21, wflat(p['b4']['wc']),
                      p['b4']['bc'].reshape(1, -1), s1, 32, 32,
                      (p['b4']['w2'], p['b4']['b2'],
                       jnp.eye(128, dtype=jnp.float32)), N2P, bn2, 128)

    y5 = _unary_call([x4], [p['b5']['w1']], p['b5']['b1'], True, bn2)
    gt5 = _tgather(_mktable([y5, pts2], 80), nb2, 67, K)
    kpd2, kpt22, s2 = consts[2]
    x5 = _kpconv_call("ws", gt5, qt2, kpd2, kpt22, wflat(p['b5']['wc']),
                      p['b5']['bc'].reshape(1, -1), s2, 64, 64,
                      (x4, p['b5']['w2'], p['b5']['b2'], p['b5']['ws'],
                       p['b5']['bs']), N2P, bn2, 256)

    u1 = _rgather(x5, up1)
    d1 = _unary_call([u1, x3], [p['d1']['w'][:256], p['d1']['w'][256:]],
                     p['d1']['b'], True, bn1)
    u0 = _rgather(d1, up0)
    out = _unary_call([u0, x1], [p['d3']['w'][:128], p['d3']['w'][128:]],
                      p['d3']['b'], False, bn0)
    return out[:N0]
